# baseline (speedup 1.0000x reference)
import functools

import jax
import jax.numpy as jnp
from jax import lax
from jax.experimental import pallas as pl
from jax.experimental.pallas import tpu as pltpu
from jax.experimental.pallas import tpu_sc as plsc

_L = 16
_TCOL = 128
_CCOLS = 4
_CHV = _CCOLS * _TCOL
_PCAP = 4096
_NPASS = 16
_REG_ROWS = 79872
_SROWS = 128
_FLUSH_AT = 112


def _make_kernel_a(B2, V, D, NC, NS):
    NW = NC * NS
    full_cols = V // _TCOL
    tail_v0 = full_cols * _TCOL
    base_cols = full_cols // NW
    last_extra = full_cols - base_cols * NW
    max_chunks = (base_cols + last_extra + _CCOLS - 1) // _CCOLS
    dump_p = B2
    mesh = plsc.VectorSubcoreMesh(core_axis_name="c", subcore_axis_name="s")

    @functools.partial(
        pl.kernel,
        mesh=mesh,
        compiler_params=pltpu.CompilerParams(
            use_tc_tiling_on_sc=True, needs_layout_passes=False),
        out_type=(
            jax.ShapeDtypeStruct((2 * _REG_ROWS * D,), jnp.float32),
            jax.ShapeDtypeStruct((2 * _REG_ROWS,), jnp.int32),
        ),
        scratch_types=[
            pltpu.VMEM((B2,), jnp.int32),
            pltpu.VMEM((_PCAP,), jnp.int32),
            pltpu.VMEM((D, _CHV), jnp.float32),
            pltpu.VMEM((D, _CHV), jnp.float32),
            pltpu.VMEM((D, _TCOL - 64), jnp.float32),
            pltpu.VMEM((_SROWS * D,), jnp.float32),
            pltpu.VMEM((_SROWS,), jnp.int32),
            pltpu.VMEM((_SROWS,), jnp.int32),
            pltpu.VMEM((_L,), jnp.int32),
            pltpu.SMEM((8,), jnp.int32),
            pltpu.SemaphoreType.DMA,
        ],
    )
    def ka(subj_hbm, obj_hbm, tabt_hbm, tail_hbm,
           ext_hbm, pos_hbm,
           idx_v, plist, chunk0, chunk1, tailbuf, estage, pstage, vstage,
           cstage, sctr, sem):
        core = lax.axis_index("c")
        sid = lax.axis_index("s")
        gw = core * NS + sid
        iota = lax.iota(jnp.int32, _L)

        @pl.when(sid == 0)
        def _():
            sctr[0] = 0

        pltpu.sync_copy(subj_hbm, idx_v.at[pl.ds(0, B2 // 2)])
        pltpu.sync_copy(obj_hbm, idx_v.at[pl.ds(B2 // 2, B2 // 2)])
        pltpu.sync_copy(tail_hbm, tailbuf)
        zeros = jnp.zeros((_L,), jnp.int32)
        dumpv = jnp.full((_L,), dump_p, jnp.int32)
        for s in range(_SROWS // _L):
            pstage[pl.ds(s * _L, _L)] = dumpv
            vstage[pl.ds(s * _L, _L)] = zeros
        for s in range(_PCAP // _L):
            plist[pl.ds(s * _L, _L)] = zeros
        pos_per_w = 2 * _REG_ROWS // NW
        for t in range(pos_per_w // _SROWS):
            pltpu.sync_copy(
                pstage,
                pos_hbm.at[pl.ds(
                    pl.multiple_of(gw * pos_per_w + t * _SROWS, _L),
                    _SROWS)])
        plsc.subcore_barrier()

        col0 = gw * base_cols
        my_cols = jnp.where(gw == NW - 1, base_cols + last_extra, base_cols)
        vlo = col0 * _TCOL
        vhi = vlo + my_cols * _TCOL
        has_tail = gw == NW - 1
        n_chunks = (my_cols + _CCOLS - 1) // _CCOLS
        region = core * _REG_ROWS

        def filter_pass(k):
            lo = k * _PCAP
            hi = lo + _PCAP

            def body(i, cnt):
                v = idx_v[pl.ds(i * _L, _L)]
                m = jnp.logical_and(v >= vlo, v < vhi)
                mt = jnp.logical_and(has_tail, v >= tail_v0)
                m = jnp.logical_or(m, mt)
                r = plsc.cumsum(jnp.where(m, 1, 0))
                ordn = cnt + r - 1
                keep = jnp.logical_and(
                    m, jnp.logical_and(ordn >= lo, ordn < hi))
                plsc.store_scatter(plist.at[:], [ordn - lo],
                                   i * _L + iota, mask=keep)
                return cnt + jnp.max(jnp.where(m, r, 0))

            return lax.fori_loop(0, B2 // _L, body, jnp.int32(0))

        def extract(src, nrows):
            def grp(g, _):
                @pl.when(g * _L < nrows)
                def _():
                    for j in range(_L):
                        row = g * _L + j
                        vb = plsc.load_gather(
                            vstage.at[:], [jnp.full((_L,), row, jnp.int32)])
                        for cb in range(D // _L):
                            vals = plsc.load_gather(
                                src.at[:, :], [cb * _L + iota, vb])
                            estage[pl.ds(row * D + cb * _L, _L)] = vals
                return 0

            lax.fori_loop(0, _SROWS // _L, grp, 0)

        def flush(nrows):
            npad = ((nrows + _L - 1) // _L) * _L
            padm = jnp.logical_and(iota >= (nrows - (npad - _L)),
                                   npad - _L + iota < npad)
            plsc.store_scatter(
                pstage.at[:], [npad - _L + iota],
                jnp.full((_L,), dump_p, jnp.int32),
                mask=jnp.logical_and(npad - _L + iota >= nrows,
                                     npad - _L + iota >= 0))
            base = plsc.fetch_and_add(sctr.at[0], npad, subcore_id=0)
            slot = pl.multiple_of(region + base, 16)
            for j in range(_SROWS // _L):
                @pl.when(j * _L < npad)
                def _():
                    pltpu.sync_copy(
                        estage.at[pl.ds(j * _L * D, _L * D)],
                        ext_hbm.at[pl.ds(
                            pl.multiple_of((slot + j * _L) * D, 1024),
                            _L * D)])
                    pltpu.sync_copy(
                        pstage.at[pl.ds(j * _L, _L)],
                        pos_hbm.at[pl.ds(
                            pl.multiple_of(slot + j * _L, 16), _L)])

        def sel_extract_flush(src, cv0, cvn, n_list):
            def sel(s, ns):
                pv = plsc.load_gather(plist.at[:], [s * _L + iota])
                vv = plsc.load_gather(idx_v.at[:], [pv])
                valid = (s * _L + iota) < n_list
                m = jnp.logical_and(valid, jnp.logical_and(
                    vv >= cv0, vv < cv0 + cvn))
                r = plsc.cumsum(jnp.where(m, 1, 0))
                dst = ns + r - 1
                plsc.store_scatter(pstage.at[:], [dst], pv, mask=m)
                plsc.store_scatter(vstage.at[:], [dst], vv - cv0, mask=m)
                ns2 = ns + jnp.max(jnp.where(m, r, 0))

                def do_flush():
                    extract(src, ns2)
                    flush(ns2)
                    return jnp.int32(0)

                return lax.cond(ns2 >= _FLUSH_AT, do_flush, lambda: ns2)

            ns_end = lax.fori_loop(0, _PCAP // _L, sel, jnp.int32(0))

            @pl.when(ns_end > 0)
            def _():
                extract(src, ns_end)
                flush(ns_end)

        total = filter_pass(0)

        def one_pass(k, _):
            active = total > k * _PCAP

            @pl.when(jnp.logical_and(active, k > 0))
            def _():
                filter_pass(k)

            n_list = jnp.clip(total - k * _PCAP, 0, _PCAP)

            @pl.when(active)
            def _():
                v0 = pl.multiple_of(vlo, _TCOL)
                pltpu.async_copy(
                    tabt_hbm.at[:, pl.ds(v0, _CHV)], chunk0, sem)

                def pair(cp, __):
                    for b, (cur, nxt) in enumerate(
                            ((chunk0, chunk1), (chunk1, chunk0))):
                        ci = cp * 2 + b

                        @pl.when(ci < n_chunks)
                        def _(ci=ci, cur=cur, nxt=nxt):
                            pltpu.make_async_copy(
                                tabt_hbm.at[:, pl.ds(0, _CHV)], cur,
                                sem).wait()

                            @pl.when(ci + 1 < n_chunks)
                            def _():
                                nx = pl.multiple_of(
                                    vlo + (ci + 1) * _CHV, _TCOL)
                                pltpu.async_copy(
                                    tabt_hbm.at[:, pl.ds(nx, _CHV)],
                                    nxt, sem)

                            sel_extract_flush(
                                cur, vlo + ci * _CHV, _CHV, n_list)
                    return 0

                lax.fori_loop(0, (max_chunks + 1) // 2, pair, 0)

                @pl.when(has_tail)
                def _():
                    sel_extract_flush(tailbuf, tail_v0, V - tail_v0, n_list)

            return 0

        lax.fori_loop(0, _NPASS, one_pass, 0)

    return ka


def _make_kernel_b(B, D, NC, NS, R_V):
    NW = NC * NS
    b_per_w = B // NW
    n_rel_chunks = b_per_w // _TCOL
    comb_rows = 2 * B + 256
    dump_row = 2 * B
    mesh = plsc.VectorSubcoreMesh(core_axis_name="c", subcore_axis_name="s")

    @functools.partial(
        pl.kernel,
        mesh=mesh,
        compiler_params=pltpu.CompilerParams(use_tc_tiling_on_sc=False),
        out_type=(
            jax.ShapeDtypeStruct((comb_rows, D), jnp.float32),
            jax.ShapeDtypeStruct((B, D), jnp.float32),
        ),
        scratch_types=[
            pltpu.VMEM((_TCOL, D), jnp.float32),
            pltpu.VMEM((1, _TCOL), jnp.int32),
            pltpu.VMEM((_TCOL,), jnp.int32),
            pltpu.VMEM((n_rel_chunks, _TCOL), jnp.int32),
            pltpu.VMEM((b_per_w, D), jnp.float32),
            pltpu.SemaphoreType.DMA,
            pltpu.SemaphoreType.DMA,
        ],
    )
    def kb(ext_hbm, pos_hbm, rel_hbm, rtab_hbm,
           comb_hbm, erel_hbm,
           rows_v, pidx, ptmp, ridx, rrows, sem, rsem):
        core = lax.axis_index("c")
        sid = lax.axis_index("s")
        gw = core * NS + sid
        iota = lax.iota(jnp.int32, _L)

        pltpu.sync_copy(rel_hbm.at[gw], ridx)
        copies = []
        for j in range(n_rel_chunks):
            copies.append(pltpu.async_copy(
                rtab_hbm.at[ridx.at[j]],
                rrows.at[pl.ds(j * _TCOL, _TCOL)], rsem))
        for c_ in copies:
            c_.wait()
        pltpu.sync_copy(rrows, erel_hbm.at[pl.ds(gw * b_per_w, b_per_w)])

        region = core * _REG_ROWS
        per_w = _REG_ROWS // NS

        def batch(b, _):
            row0 = pl.multiple_of(region + sid * per_w + b * _TCOL, _TCOL)
            pltpu.sync_copy(pos_hbm.at[pl.ds(row0, _TCOL)], ptmp)
            pltpu.sync_copy(ext_hbm.at[pl.ds(row0, _TCOL)], rows_v)
            for s in range(_TCOL // _L):
                pv = ptmp[pl.ds(s * _L, _L)]
                valid = jnp.logical_and(pv >= 0, pv < dump_row)
                pidx[0, pl.ds(s * _L, _L)] = jnp.where(valid, pv, dump_row)
            pltpu.async_copy(rows_v, comb_hbm.at[pidx.at[0]], sem).wait()
            return 0

        lax.fori_loop(0, per_w // _TCOL, batch, 0)

    return kb


def kernel(subj, rel, obj, entity_table, relation_table):
    B = subj.shape[0]
    V, D = entity_table.shape
    info = plsc.get_sparse_core_info()
    NC, NS = info.num_cores, info.num_subcores
    NW = NC * NS

    tabt = entity_table.T
    full_cols = V // _TCOL
    tail_v0 = full_cols * _TCOL
    tail = entity_table[tail_v0:].T

    ka = _make_kernel_a(2 * B, V, D, NC, NS)
    ext1d, pos = ka(subj, obj, tabt, tail)

    ext2d = ext1d.reshape(-1, D)
    rel_r = rel.reshape(NW, -1, _TCOL)
    kb = _make_kernel_b(B, D, NC, NS, relation_table.shape[0])
    comb, e_rel = kb(ext2d, pos, rel_r, relation_table)

    e_subj = comb[:B]
    e_obj = comb[B:2 * B]
    return (e_subj, e_rel, e_obj)


# speedup vs baseline: 1.1764x; 1.1764x over previous
import functools

import jax
import jax.numpy as jnp
from jax import lax
from jax.experimental import pallas as pl
from jax.experimental.pallas import tpu as pltpu
from jax.experimental.pallas import tpu_sc as plsc

_L = 16
_TCOL = 128
_CCOLS = 4
_CHV = _CCOLS * _TCOL
_PCAP = 4096
_NPASS = 16
_REG_ROWS = 79872
_SROWS = 128
_FLUSH_AT = 112


def _make_kernel_a(B2, V, D, NC, NS):
    NW = NC * NS
    full_cols = V // _TCOL
    tail_v0 = full_cols * _TCOL
    base_cols = full_cols // NW
    last_extra = full_cols - base_cols * NW
    max_chunks = (base_cols + last_extra + _CCOLS - 1) // _CCOLS
    dump_p = B2
    mesh = plsc.VectorSubcoreMesh(core_axis_name="c", subcore_axis_name="s")

    @functools.partial(
        pl.kernel,
        mesh=mesh,
        compiler_params=pltpu.CompilerParams(
            use_tc_tiling_on_sc=True, needs_layout_passes=False),
        out_type=(
            jax.ShapeDtypeStruct((2 * _REG_ROWS * D,), jnp.float32),
            jax.ShapeDtypeStruct((2 * _REG_ROWS,), jnp.int32),
        ),
        scratch_types=[
            pltpu.VMEM((B2,), jnp.int32),
            pltpu.VMEM((_PCAP,), jnp.int32),
            pltpu.VMEM((_PCAP,), jnp.int32),
            pltpu.VMEM((_L,), jnp.int32),
            pltpu.VMEM((_L,), jnp.int32),
            pltpu.VMEM((D, _CHV), jnp.float32),
            pltpu.VMEM((D, _CHV), jnp.float32),
            pltpu.VMEM((D, _TCOL - 64), jnp.float32),
            pltpu.VMEM((_SROWS * D,), jnp.float32),
            pltpu.VMEM((_SROWS,), jnp.int32),
            pltpu.VMEM((_SROWS,), jnp.int32),
            pltpu.VMEM((_L,), jnp.int32),
            pltpu.SMEM((8,), jnp.int32),
            pltpu.SemaphoreType.DMA,
        ],
    )
    def ka(subj_hbm, obj_hbm, tabt_hbm, tail_hbm,
           ext_hbm, pos_hbm,
           idx_v, plist, vlist, tmpp, tmpv, chunk0, chunk1, tailbuf,
           estage, pstage, vstage, cstage, sctr, sem):
        core = lax.axis_index("c")
        sid = lax.axis_index("s")
        gw = core * NS + sid
        iota = lax.iota(jnp.int32, _L)

        @pl.when(sid == 0)
        def _():
            sctr[0] = 0

        pltpu.sync_copy(subj_hbm, idx_v.at[pl.ds(0, B2 // 2)])
        pltpu.sync_copy(obj_hbm, idx_v.at[pl.ds(B2 // 2, B2 // 2)])
        pltpu.sync_copy(tail_hbm, tailbuf)
        zeros = jnp.zeros((_L,), jnp.int32)
        dumpv = jnp.full((_L,), dump_p, jnp.int32)
        for s in range(_SROWS // _L):
            pstage[pl.ds(s * _L, _L)] = dumpv
            vstage[pl.ds(s * _L, _L)] = zeros
        for s in range(_PCAP // _L):
            plist[pl.ds(s * _L, _L)] = zeros
        pos_per_w = 2 * _REG_ROWS // NW
        for t in range(pos_per_w // _SROWS):
            pltpu.sync_copy(
                pstage,
                pos_hbm.at[pl.ds(
                    pl.multiple_of(gw * pos_per_w + t * _SROWS, _L),
                    _SROWS)])
        plsc.subcore_barrier()

        col0 = gw * base_cols
        my_cols = jnp.where(gw == NW - 1, base_cols + last_extra, base_cols)
        vlo = col0 * _TCOL
        vhi = vlo + my_cols * _TCOL
        has_tail = gw == NW - 1
        n_chunks = (my_cols + _CCOLS - 1) // _CCOLS
        region = core * _REG_ROWS

        def filter_pass(k):
            lo = k * _PCAP
            hi = lo + _PCAP

            def body(i, ns):
                v = idx_v[pl.ds(i * _L, _L)]
                m = jnp.logical_and(v >= vlo, v < vhi)
                mt = jnp.logical_and(has_tail, v >= tail_v0)
                m = jnp.logical_or(m, mt)
                plsc.store_compressed(tmpp.at[:], i * _L + iota, mask=m)
                plsc.store_compressed(tmpv.at[:], v, mask=m)
                n = plsc.all_reduce_population_count(m)
                ordn = ns + iota
                sm = jnp.logical_and(
                    iota < n, jnp.logical_and(ordn >= lo, ordn < hi))
                tp = tmpp[...]
                tv = tmpv[...]
                plsc.store_scatter(plist.at[:], [ordn - lo], tp, mask=sm)
                plsc.store_scatter(vlist.at[:], [ordn - lo], tv, mask=sm)
                return ns + n

            ns_end = lax.fori_loop(
                0, B2 // _L, body, jnp.zeros((_L,), jnp.int32))
            return jnp.max(ns_end)

        def extract(src, nrows):
            def grp(g, _):
                @pl.when(g * _L < nrows)
                def _():
                    for j in range(_L):
                        row = g * _L + j
                        vb = plsc.load_gather(
                            vstage.at[:], [jnp.full((_L,), row, jnp.int32)])
                        for cb in range(D // _L):
                            vals = plsc.load_gather(
                                src.at[:, :], [cb * _L + iota, vb])
                            estage[pl.ds(row * D + cb * _L, _L)] = vals
                return 0

            lax.fori_loop(0, _SROWS // _L, grp, 0)

        def flush(nrows):
            npad = ((nrows + _L - 1) // _L) * _L
            padm = jnp.logical_and(iota >= (nrows - (npad - _L)),
                                   npad - _L + iota < npad)
            plsc.store_scatter(
                pstage.at[:], [npad - _L + iota],
                jnp.full((_L,), dump_p, jnp.int32),
                mask=jnp.logical_and(npad - _L + iota >= nrows,
                                     npad - _L + iota >= 0))
            base = plsc.fetch_and_add(sctr.at[0], npad, subcore_id=0)
            slot = pl.multiple_of(region + base, 16)
            for j in range(_SROWS // _L):
                @pl.when(j * _L < npad)
                def _():
                    pltpu.sync_copy(
                        estage.at[pl.ds(j * _L * D, _L * D)],
                        ext_hbm.at[pl.ds(
                            pl.multiple_of((slot + j * _L) * D, 1024),
                            _L * D)])
                    pltpu.sync_copy(
                        pstage.at[pl.ds(j * _L, _L)],
                        pos_hbm.at[pl.ds(
                            pl.multiple_of(slot + j * _L, 16), _L)])

        def sel_extract_flush(src, cv0, cvn, n_list):
            def sel(s, ns):
                pv = plist[pl.ds(s * _L, _L)]
                vv = vlist[pl.ds(s * _L, _L)]
                valid = (s * _L + iota) < n_list
                m = jnp.logical_and(valid, jnp.logical_and(
                    vv >= cv0, vv < cv0 + cvn))
                plsc.store_compressed(tmpp.at[:], pv, mask=m)
                plsc.store_compressed(tmpv.at[:], vv - cv0, mask=m)
                n = plsc.all_reduce_population_count(m)
                sm = iota < n
                tp = tmpp[...]
                tv = tmpv[...]
                plsc.store_scatter(pstage.at[:], [ns + iota], tp, mask=sm)
                plsc.store_scatter(vstage.at[:], [ns + iota], tv, mask=sm)
                ns2 = ns + n

                def do_flush():
                    nr = jnp.max(ns2)
                    extract(src, nr)
                    flush(nr)
                    return jnp.zeros((_L,), jnp.int32)

                return lax.cond(jnp.any(ns2 >= _FLUSH_AT), do_flush,
                                lambda: ns2)

            nvregs = (n_list + _L - 1) // _L
            ns_end = lax.fori_loop(0, nvregs, sel,
                                   jnp.zeros((_L,), jnp.int32))
            nr_end = jnp.max(ns_end)

            @pl.when(nr_end > 0)
            def _():
                extract(src, nr_end)
                flush(nr_end)

        total = filter_pass(0)

        def one_pass(k, _):
            active = total > k * _PCAP

            @pl.when(jnp.logical_and(active, k > 0))
            def _():
                filter_pass(k)

            n_list = jnp.clip(total - k * _PCAP, 0, _PCAP)

            @pl.when(active)
            def _():
                v0 = pl.multiple_of(vlo, _TCOL)
                pltpu.async_copy(
                    tabt_hbm.at[:, pl.ds(v0, _CHV)], chunk0, sem)

                def pair(cp, __):
                    for b, (cur, nxt) in enumerate(
                            ((chunk0, chunk1), (chunk1, chunk0))):
                        ci = cp * 2 + b

                        @pl.when(ci < n_chunks)
                        def _(ci=ci, cur=cur, nxt=nxt):
                            pltpu.make_async_copy(
                                tabt_hbm.at[:, pl.ds(0, _CHV)], cur,
                                sem).wait()

                            @pl.when(ci + 1 < n_chunks)
                            def _():
                                nx = pl.multiple_of(
                                    vlo + (ci + 1) * _CHV, _TCOL)
                                pltpu.async_copy(
                                    tabt_hbm.at[:, pl.ds(nx, _CHV)],
                                    nxt, sem)

                            sel_extract_flush(
                                cur, vlo + ci * _CHV, _CHV, n_list)
                    return 0

                lax.fori_loop(0, (max_chunks + 1) // 2, pair, 0)

                @pl.when(has_tail)
                def _():
                    sel_extract_flush(tailbuf, tail_v0, V - tail_v0, n_list)

            return 0

        lax.fori_loop(0, _NPASS, one_pass, 0)

    return ka


def _make_kernel_b(B, D, NC, NS, R_V):
    NW = NC * NS
    b_per_w = B // NW
    n_rel_chunks = b_per_w // _TCOL
    comb_rows = 2 * B + 256
    dump_row = 2 * B
    mesh = plsc.VectorSubcoreMesh(core_axis_name="c", subcore_axis_name="s")

    @functools.partial(
        pl.kernel,
        mesh=mesh,
        compiler_params=pltpu.CompilerParams(use_tc_tiling_on_sc=False),
        out_type=(
            jax.ShapeDtypeStruct((comb_rows, D), jnp.float32),
            jax.ShapeDtypeStruct((B, D), jnp.float32),
        ),
        scratch_types=[
            pltpu.VMEM((_TCOL, D), jnp.float32),
            pltpu.VMEM((1, _TCOL), jnp.int32),
            pltpu.VMEM((_TCOL,), jnp.int32),
            pltpu.VMEM((n_rel_chunks, _TCOL), jnp.int32),
            pltpu.VMEM((b_per_w, D), jnp.float32),
            pltpu.SemaphoreType.DMA,
            pltpu.SemaphoreType.DMA,
        ],
    )
    def kb(ext_hbm, pos_hbm, rel_hbm, rtab_hbm,
           comb_hbm, erel_hbm,
           rows_v, pidx, ptmp, ridx, rrows, sem, rsem):
        core = lax.axis_index("c")
        sid = lax.axis_index("s")
        gw = core * NS + sid
        iota = lax.iota(jnp.int32, _L)

        pltpu.sync_copy(rel_hbm.at[gw], ridx)
        copies = []
        for j in range(n_rel_chunks):
            copies.append(pltpu.async_copy(
                rtab_hbm.at[ridx.at[j]],
                rrows.at[pl.ds(j * _TCOL, _TCOL)], rsem))
        for c_ in copies:
            c_.wait()
        pltpu.sync_copy(rrows, erel_hbm.at[pl.ds(gw * b_per_w, b_per_w)])

        region = core * _REG_ROWS
        per_w = _REG_ROWS // NS

        def batch(b, _):
            row0 = pl.multiple_of(region + sid * per_w + b * _TCOL, _TCOL)
            pltpu.sync_copy(pos_hbm.at[pl.ds(row0, _TCOL)], ptmp)
            pltpu.sync_copy(ext_hbm.at[pl.ds(row0, _TCOL)], rows_v)
            for s in range(_TCOL // _L):
                pv = ptmp[pl.ds(s * _L, _L)]
                valid = jnp.logical_and(pv >= 0, pv < dump_row)
                pidx[0, pl.ds(s * _L, _L)] = jnp.where(valid, pv, dump_row)
            pltpu.async_copy(rows_v, comb_hbm.at[pidx.at[0]], sem).wait()
            return 0

        lax.fori_loop(0, per_w // _TCOL, batch, 0)

    return kb


def kernel(subj, rel, obj, entity_table, relation_table):
    B = subj.shape[0]
    V, D = entity_table.shape
    info = plsc.get_sparse_core_info()
    NC, NS = info.num_cores, info.num_subcores
    NW = NC * NS

    tabt = entity_table.T
    full_cols = V // _TCOL
    tail_v0 = full_cols * _TCOL
    tail = entity_table[tail_v0:].T

    ka = _make_kernel_a(2 * B, V, D, NC, NS)
    ext1d, pos = ka(subj, obj, tabt, tail)

    ext2d = ext1d.reshape(-1, D)
    rel_r = rel.reshape(NW, -1, _TCOL)
    kb = _make_kernel_b(B, D, NC, NS, relation_table.shape[0])
    comb, e_rel = kb(ext2d, pos, rel_r, relation_table)

    e_subj = comb[:B]
    e_obj = comb[B:2 * B]
    return (e_subj, e_rel, e_obj)


# speedup vs baseline: 6.1456x; 5.2239x over previous
import functools

import jax
import jax.numpy as jnp
from jax import lax
from jax.experimental import pallas as pl
from jax.experimental.pallas import tpu as pltpu
from jax.experimental.pallas import tpu_sc as plsc

_L = 16
_RING = 16
_OROWS = 128


def _make_kernel(B, D, NC, NS):
    NW = NC * NS
    bpw = B // NW
    mesh = plsc.VectorSubcoreMesh(core_axis_name="c", subcore_axis_name="s")

    @functools.partial(
        pl.kernel,
        mesh=mesh,
        compiler_params=pltpu.CompilerParams(
            use_tc_tiling_on_sc=True, needs_layout_passes=False),
        out_type=(
            jax.ShapeDtypeStruct((B, D), jnp.float32),
            jax.ShapeDtypeStruct((B, D), jnp.float32),
            jax.ShapeDtypeStruct((B, D), jnp.float32),
        ),
        scratch_types=[
            pltpu.VMEM((3 * 512,), jnp.int32),
            pltpu.VMEM((_RING, 8, D), jnp.float32),
            pltpu.VMEM((_OROWS, D), jnp.float32),
            pltpu.SemaphoreType.DMA,
            pltpu.SemaphoreType.DMA,
        ],
    )
    def k(subj_hbm, rel_hbm, obj_hbm, etab_hbm, rtab_hbm,
          o_subj, o_rel, o_obj,
          idx_v, ring, ostage, sem, osem):
        core = lax.axis_index("c")
        sid = lax.axis_index("s")
        gw = core * NS + sid
        base = gw * bpw
        iota = lax.iota(jnp.int32, _L)

        pltpu.sync_copy(subj_hbm.at[pl.ds(base, bpw)], idx_v.at[pl.ds(0, bpw)])
        pltpu.sync_copy(rel_hbm.at[pl.ds(base, bpw)],
                        idx_v.at[pl.ds(bpw, bpw)])
        pltpu.sync_copy(obj_hbm.at[pl.ds(base, bpw)],
                        idx_v.at[pl.ds(2 * bpw, bpw)])

        def do_table(tab, out, ioff):
            def block(blk, _):
                i0 = ioff + blk * _OROWS

                def group(g, __):
                    vs = []
                    cps = []
                    for j in range(_RING):
                        v = jnp.max(plsc.load_gather(
                            idx_v.at[:],
                            [jnp.full((_L,), i0 + g * _RING + j, jnp.int32)]))
                        t0 = pl.multiple_of(8 * (v // 8), 8)
                        cps.append(pltpu.async_copy(
                            tab.at[pl.ds(t0, 8), :], ring.at[j], sem))
                        vs.append(v)
                    for c in cps:
                        c.wait()
                    for j in range(_RING):
                        r = vs[j] % 8
                        row = g * _RING + j
                        for cb in range(D // _L):
                            vals = ring[j, r, pl.ds(cb * _L, _L)]
                            ostage[row, pl.ds(cb * _L, _L)] = vals
                    return 0

                lax.fori_loop(0, _OROWS // _RING, group, 0)
                pltpu.sync_copy(
                    ostage,
                    out.at[pl.ds(base + blk * _OROWS, _OROWS)])
                return 0

            lax.fori_loop(0, bpw // _OROWS, block, 0)

        do_table(etab_hbm, o_subj, 0)
        do_table(rtab_hbm, o_rel, bpw)
        do_table(etab_hbm, o_obj, 2 * bpw)

    return k


def kernel(subj, rel, obj, entity_table, relation_table):
    B = subj.shape[0]
    D = entity_table.shape[1]
    info = plsc.get_sparse_core_info()
    k = _make_kernel(B, D, info.num_cores, info.num_subcores)
    return k(subj, rel, obj, entity_table, relation_table)


# speedup vs baseline: 6.5839x; 1.0713x over previous
import functools

import jax
import jax.numpy as jnp
from jax import lax
from jax.experimental import pallas as pl
from jax.experimental.pallas import tpu as pltpu
from jax.experimental.pallas import tpu_sc as plsc

_L = 16
_RING = 16
_OROWS = 128


def _make_kernel(B, D, NC, NS):
    NW = NC * NS
    bpw = B // NW
    mesh = plsc.VectorSubcoreMesh(core_axis_name="c", subcore_axis_name="s")

    @functools.partial(
        pl.kernel,
        mesh=mesh,
        compiler_params=pltpu.CompilerParams(
            use_tc_tiling_on_sc=True, needs_layout_passes=False),
        out_type=(
            jax.ShapeDtypeStruct((B, D), jnp.float32),
            jax.ShapeDtypeStruct((B, D), jnp.float32),
            jax.ShapeDtypeStruct((B, D), jnp.float32),
        ),
        scratch_types=[
            pltpu.VMEM((3 * 512,), jnp.int32),
            pltpu.VMEM((2, _RING, 8, D), jnp.float32),
            pltpu.VMEM((_OROWS, D), jnp.float32),
            pltpu.SemaphoreType.DMA,
            pltpu.SemaphoreType.DMA,
        ],
    )
    def k(subj_hbm, rel_hbm, obj_hbm, etab_hbm, rtab_hbm,
          o_subj, o_rel, o_obj,
          idx_v, ring, ostage, sem, osem):
        core = lax.axis_index("c")
        sid = lax.axis_index("s")
        gw = core * NS + sid
        base = gw * bpw
        iota = lax.iota(jnp.int32, _L)

        pltpu.sync_copy(subj_hbm.at[pl.ds(base, bpw)], idx_v.at[pl.ds(0, bpw)])
        pltpu.sync_copy(rel_hbm.at[pl.ds(base, bpw)],
                        idx_v.at[pl.ds(bpw, bpw)])
        pltpu.sync_copy(obj_hbm.at[pl.ds(base, bpw)],
                        idx_v.at[pl.ds(2 * bpw, bpw)])

        def do_table(tab, out, ioff):
            ngrp = _OROWS // _RING

            def lookup_scalar(i):
                return jnp.max(plsc.load_gather(
                    idx_v.at[:], [jnp.full((_L,), i, jnp.int32)]))

            def issue_group(i0, g, rb):
                for j in range(_RING):
                    v = lookup_scalar(i0 + g * _RING + j)
                    t0 = pl.multiple_of(8 * (v // 8), 8)
                    pltpu.async_copy(
                        tab.at[pl.ds(t0, 8), :], ring.at[rb, j], sem)

            def extract_group(i0, g, rb):
                for j in range(_RING):
                    pltpu.make_async_copy(
                        tab.at[pl.ds(0, 8), :], ring.at[rb, j], sem).wait()
                for j in range(_RING):
                    v = lookup_scalar(i0 + g * _RING + j)
                    r = v % 8
                    row = g * _RING + j
                    for cb in range(D // _L):
                        vals = ring[rb, j, r, pl.ds(cb * _L, _L)]
                        ostage[row, pl.ds(cb * _L, _L)] = vals

            def block(blk, _):
                i0 = ioff + blk * _OROWS
                issue_group(i0, 0, 0)

                def pair(p, __):
                    issue_group(i0, 2 * p + 1, 1)
                    extract_group(i0, 2 * p, 0)

                    @pl.when(p < ngrp // 2 - 1)
                    def _():
                        issue_group(i0, 2 * p + 2, 0)
                    extract_group(i0, 2 * p + 1, 1)
                    return 0

                lax.fori_loop(0, ngrp // 2, pair, 0)
                pltpu.sync_copy(
                    ostage,
                    out.at[pl.ds(base + blk * _OROWS, _OROWS)])
                return 0

            lax.fori_loop(0, bpw // _OROWS, block, 0)

        do_table(etab_hbm, o_subj, 0)
        do_table(rtab_hbm, o_rel, bpw)
        do_table(etab_hbm, o_obj, 2 * bpw)

    return k


def kernel(subj, rel, obj, entity_table, relation_table):
    B = subj.shape[0]
    D = entity_table.shape[1]
    info = plsc.get_sparse_core_info()
    k = _make_kernel(B, D, info.num_cores, info.num_subcores)
    return k(subj, rel, obj, entity_table, relation_table)
